# baseline (speedup 1.0000x reference)
import jax
import jax.numpy as jnp
from jax.experimental import pallas as pl
import jax.experimental.pallas.tpu as pltpu


def _prep_kernel(x_ref, s_ref, rsum_ref, acc_ref):
    j = pl.program_id(1)
    nj = pl.num_programs(1)
    s = jnp.exp(x_ref[...] * 50.0)
    s_ref[...] = s
    part = jnp.sum(s, axis=1, keepdims=True)

    @pl.when(j == 0)
    def _():
        acc_ref[...] = part

    @pl.when(j != 0)
    def _():
        acc_ref[...] += part

    @pl.when(j == nj - 1)
    def _():
        rsum_ref[...] = acc_ref[...]


def _make_mega_kernel(k_res):
    def _mega_kernel(simst_ref, sbot_ref, bsum_ref, out_ref, idx_ref,
                     stop_ref, r_ref, racc_ref, c_ref, bi_ref):
        bv_ref = racc_ref
        p = pl.program_id(0)
        j = pl.program_id(1)
        npass = pl.num_programs(0)
        nj = pl.num_programs(1)
        cb = sbot_ref.shape[1]
        ds = pl.ds(j * cb, cb)

        @pl.when(p == 0)
        def _():
            st = jnp.exp(simst_ref[...] * 50.0)
            stop_ref[:, ds] = st
            part = jnp.sum(st, axis=1, keepdims=True)

            @pl.when(j == 0)
            def _():
                racc_ref[:k_res] = part

            @pl.when(j != 0)
            def _():
                racc_ref[:k_res] += part

            @pl.when(j == nj - 1)
            def _():
                r_ref[:k_res] = 1.0 / racc_ref[:k_res]
                r_ref[k_res:] = 1.0 / bsum_ref[...]

        @pl.when((p % 2 == 1) & (p < npass - 1))
        def _():
            t = (jnp.sum(stop_ref[:, ds] * r_ref[:k_res], axis=0, keepdims=True)
                 + jnp.sum(sbot_ref[...] * r_ref[k_res:], axis=0, keepdims=True))
            c_ref[:, ds] = 1.0 / t

        @pl.when((p % 2 == 0) & (p > 0))
        def _():
            cj = c_ref[:, ds]
            pt = jnp.sum(stop_ref[:, ds] * cj, axis=1, keepdims=True)
            pb = jnp.sum(sbot_ref[...] * cj, axis=1, keepdims=True)

            @pl.when(j == 0)
            def _():
                racc_ref[:k_res] = pt
                racc_ref[k_res:] = pb

            @pl.when(j != 0)
            def _():
                racc_ref[:k_res] += pt
                racc_ref[k_res:] += pb

            @pl.when(j == nj - 1)
            def _():
                r_ref[...] = 1.0 / racc_ref[...]

        @pl.when(p == npass - 1)
        def _():
            st = stop_ref[:, ds] * r_ref[:k_res]
            sb = sbot_ref[...] * r_ref[k_res:]
            t = (jnp.sum(st, axis=0, keepdims=True)
                 + jnp.sum(sb, axis=0, keepdims=True))
            cinv = 1.0 / t
            ot = st * cinv
            ob = sb * cinv
            out_ref[:k_res, :] = ot
            out_ref[k_res:, :] = ob
            kt = stop_ref.shape[0]
            mb = sbot_ref.shape[0]
            bmt = jnp.max(ot, axis=1, keepdims=True)
            bit = jnp.argmax(ot, axis=1).reshape(kt, 1).astype(jnp.int32) + j * cb
            bmb = jnp.max(ob, axis=1, keepdims=True)
            bib = jnp.argmax(ob, axis=1).reshape(mb, 1).astype(jnp.int32) + j * cb

            @pl.when(j == 0)
            def _():
                bv_ref[:k_res] = bmt
                bv_ref[k_res:] = bmb
                bi_ref[:k_res] = bit
                bi_ref[k_res:] = bib

            @pl.when(j != 0)
            def _():
                ut = bmt > bv_ref[:k_res]
                bv_ref[:k_res] = jnp.where(ut, bmt, bv_ref[:k_res])
                bi_ref[:k_res] = jnp.where(ut, bit, bi_ref[:k_res])
                ub = bmb > bv_ref[k_res:]
                bv_ref[k_res:] = jnp.where(ub, bmb, bv_ref[k_res:])
                bi_ref[k_res:] = jnp.where(ub, bib, bi_ref[k_res:])

            @pl.when(j == nj - 1)
            def _():
                idx_ref[...] = bi_ref[...].T

    return _mega_kernel


def kernel(sims, batch_size=256):
    del batch_size
    num_row, num_col = sims.shape
    work = sims.T if num_row >= num_col else sims
    m, n = work.shape

    cb = min(256, n)
    k_res = (m // 4) // 256 * 256
    k_res = max(k_res, min(m, 256))
    mb = m - k_res
    rbp = 256
    npass = 10

    s_bot, bsum = pl.pallas_call(
        _prep_kernel,
        grid=(mb // rbp, n // (2 * cb)),
        in_specs=[pl.BlockSpec((rbp, 2 * cb),
                               lambda i, j, K=k_res // rbp: (i + K, j))],
        out_specs=[
            pl.BlockSpec((rbp, 2 * cb), lambda i, j: (i, j)),
            pl.BlockSpec((rbp, 1), lambda i, j: (i, 0)),
        ],
        out_shape=[
            jax.ShapeDtypeStruct((mb, n), jnp.float32),
            jax.ShapeDtypeStruct((mb, 1), jnp.float32),
        ],
        scratch_shapes=[pltpu.VMEM((rbp, 1), jnp.float32)],
    )(work)

    out, idx = pl.pallas_call(
        _make_mega_kernel(k_res),
        grid=(npass, n // cb),
        in_specs=[
            pl.BlockSpec((k_res, cb), lambda p, j: (0, jnp.where(p == 0, j, 0))),
            pl.BlockSpec((mb, cb), lambda p, j: (0, jnp.where(p == 0, 0, j))),
            pl.BlockSpec((mb, 1), lambda p, j: (0, 0)),
        ],
        out_specs=[
            pl.BlockSpec((m, cb),
                         lambda p, j, P=npass - 1: (0, jnp.where(p == P, j, 0))),
            pl.BlockSpec((1, m), lambda p, j: (0, 0)),
        ],
        out_shape=[
            jax.ShapeDtypeStruct((m, n), jnp.float32),
            jax.ShapeDtypeStruct((1, m), jnp.int32),
        ],
        scratch_shapes=[
            pltpu.VMEM((k_res, n), jnp.float32),
            pltpu.VMEM((m, 1), jnp.float32),
            pltpu.VMEM((m, 1), jnp.float32),
            pltpu.VMEM((1, n), jnp.float32),
            pltpu.VMEM((m, 1), jnp.int32),
        ],
        compiler_params=pltpu.CompilerParams(
            vmem_limit_bytes=64 * 1024 * 1024,
        ),
    )(work, s_bot, bsum)

    row_ids = jnp.arange(m, dtype=jnp.int32)
    col_ids = idx.reshape(m)
    if num_row >= num_col:
        indices = jnp.stack((col_ids, row_ids), axis=0)
    else:
        indices = jnp.stack((row_ids, col_ids), axis=0)
    values = jnp.ones((m,), dtype=jnp.float32)
    return (out, indices, values)


# speedup vs baseline: 1.0150x; 1.0150x over previous
import jax
import jax.numpy as jnp
from jax.experimental import pallas as pl
import jax.experimental.pallas.tpu as pltpu


def _prep_kernel(x_ref, s_ref, rsum_ref, acc_ref):
    j = pl.program_id(1)
    nj = pl.num_programs(1)
    s = jnp.exp(x_ref[...] * 50.0)
    s_ref[...] = s
    part = jnp.sum(s, axis=1, keepdims=True)

    @pl.when(j == 0)
    def _():
        acc_ref[...] = part

    @pl.when(j != 0)
    def _():
        acc_ref[...] += part

    @pl.when(j == nj - 1)
    def _():
        rsum_ref[...] = acc_ref[...].T


def _make_mega_kernel(k_res, cb):
    nh = cb // 128

    def _mega_kernel(simst_ref, sbot_ref, bsum_ref, out_ref, idx_ref,
                     stop_ref, r_ref, racc_ref, c_ref, bi_ref):
        p = pl.program_id(0)
        j = pl.program_id(1)
        npass = pl.num_programs(0)
        nj = pl.num_programs(1)

        @pl.when(p == 0)
        def _():
            st = jnp.exp(simst_ref[...] * 50.0)
            stop_ref[j] = st

            for h in range(nh):
                ph = st[:, h * 128:(h + 1) * 128]
                if h == 0:
                    @pl.when(j == 0)
                    def _():
                        racc_ref[:k_res] = ph

                    @pl.when(j != 0)
                    def _():
                        racc_ref[:k_res] += ph
                else:
                    racc_ref[:k_res] += ph

            @pl.when(j == nj - 1)
            def _():
                r_ref[:k_res] = 1.0 / jnp.sum(racc_ref[:k_res], axis=1,
                                              keepdims=True)
                r_ref[k_res:] = (1.0 / bsum_ref[...]).T

        @pl.when((p % 2 == 1) & (p < npass - 1))
        def _():
            t = (jnp.sum(stop_ref[j] * r_ref[:k_res], axis=0, keepdims=True)
                 + jnp.sum(sbot_ref[...] * r_ref[k_res:], axis=0,
                           keepdims=True))
            c_ref[j] = 1.0 / t

        @pl.when((p % 2 == 0) & (p > 0))
        def _():
            cj = c_ref[j]
            st = stop_ref[j] * cj
            sb = sbot_ref[...] * cj
            for h in range(nh):
                sth = st[:, h * 128:(h + 1) * 128]
                sbh = sb[:, h * 128:(h + 1) * 128]
                if h == 0:
                    @pl.when(j == 0)
                    def _():
                        racc_ref[:k_res] = sth
                        racc_ref[k_res:] = sbh

                    @pl.when(j != 0)
                    def _():
                        racc_ref[:k_res] += sth
                        racc_ref[k_res:] += sbh
                else:
                    racc_ref[:k_res] += sth
                    racc_ref[k_res:] += sbh

            @pl.when(j == nj - 1)
            def _():
                r_ref[...] = 1.0 / jnp.sum(racc_ref[...], axis=1,
                                           keepdims=True)

        @pl.when(p == npass - 1)
        def _():
            bv_ref = racc_ref
            st = stop_ref[j] * r_ref[:k_res]
            sb = sbot_ref[...] * r_ref[k_res:]
            t = (jnp.sum(st, axis=0, keepdims=True)
                 + jnp.sum(sb, axis=0, keepdims=True))
            cinv = 1.0 / t
            ot = st * cinv
            ob = sb * cinv
            out_ref[:k_res, :] = ot
            out_ref[k_res:, :] = ob
            kt = ot.shape[0]
            mb = ob.shape[0]
            bmt = jnp.max(ot, axis=1, keepdims=True)
            bit = jnp.argmax(ot, axis=1).reshape(kt, 1).astype(jnp.int32) + j * cb
            bmb = jnp.max(ob, axis=1, keepdims=True)
            bib = jnp.argmax(ob, axis=1).reshape(mb, 1).astype(jnp.int32) + j * cb

            @pl.when(j == 0)
            def _():
                bv_ref[:k_res, 0:1] = bmt
                bv_ref[k_res:, 0:1] = bmb
                bi_ref[:k_res] = bit
                bi_ref[k_res:] = bib

            @pl.when(j != 0)
            def _():
                ut = bmt > bv_ref[:k_res, 0:1]
                bv_ref[:k_res, 0:1] = jnp.where(ut, bmt, bv_ref[:k_res, 0:1])
                bi_ref[:k_res] = jnp.where(ut, bit, bi_ref[:k_res])
                ub = bmb > bv_ref[k_res:, 0:1]
                bv_ref[k_res:, 0:1] = jnp.where(ub, bmb, bv_ref[k_res:, 0:1])
                bi_ref[k_res:] = jnp.where(ub, bib, bi_ref[k_res:])

            @pl.when(j == nj - 1)
            def _():
                idx_ref[...] = bi_ref[...].T

    return _mega_kernel


def kernel(sims, batch_size=256):
    del batch_size
    num_row, num_col = sims.shape
    work = sims.T if num_row >= num_col else sims
    m, n = work.shape

    cb = min(256, n)
    k_res = (m // 4) // 256 * 256
    k_res = max(k_res, min(m, 256))
    mb = m - k_res
    rbp = 256
    npass = 10
    nj = n // cb

    s_bot, bsum = pl.pallas_call(
        _prep_kernel,
        grid=(mb // rbp, n // (2 * cb)),
        in_specs=[pl.BlockSpec((rbp, 2 * cb),
                               lambda i, j, K=k_res // rbp: (i + K, j))],
        out_specs=[
            pl.BlockSpec((rbp, 2 * cb), lambda i, j: (i, j)),
            pl.BlockSpec((1, rbp), lambda i, j: (0, i)),
        ],
        out_shape=[
            jax.ShapeDtypeStruct((mb, n), jnp.float32),
            jax.ShapeDtypeStruct((1, mb), jnp.float32),
        ],
        scratch_shapes=[pltpu.VMEM((rbp, 1), jnp.float32)],
    )(work)

    out, idx = pl.pallas_call(
        _make_mega_kernel(k_res, cb),
        grid=(npass, nj),
        in_specs=[
            pl.BlockSpec((k_res, cb), lambda p, j: (0, jnp.where(p == 0, j, 0))),
            pl.BlockSpec((mb, cb), lambda p, j: (0, jnp.where(p == 0, 0, j))),
            pl.BlockSpec((1, mb), lambda p, j: (0, 0)),
        ],
        out_specs=[
            pl.BlockSpec((m, cb),
                         lambda p, j, P=npass - 1: (0, jnp.where(p == P, j, 0))),
            pl.BlockSpec((1, m), lambda p, j: (0, 0)),
        ],
        out_shape=[
            jax.ShapeDtypeStruct((m, n), jnp.float32),
            jax.ShapeDtypeStruct((1, m), jnp.int32),
        ],
        scratch_shapes=[
            pltpu.VMEM((nj, k_res, cb), jnp.float32),
            pltpu.VMEM((m, 1), jnp.float32),
            pltpu.VMEM((m, 128), jnp.float32),
            pltpu.VMEM((nj, 1, cb), jnp.float32),
            pltpu.VMEM((m, 1), jnp.int32),
        ],
        compiler_params=pltpu.CompilerParams(
            vmem_limit_bytes=64 * 1024 * 1024,
        ),
    )(work, s_bot, bsum)

    row_ids = jnp.arange(m, dtype=jnp.int32)
    col_ids = idx.reshape(m)
    if num_row >= num_col:
        indices = jnp.stack((col_ids, row_ids), axis=0)
    else:
        indices = jnp.stack((row_ids, col_ids), axis=0)
    values = jnp.ones((m,), dtype=jnp.float32)
    return (out, indices, values)


# speedup vs baseline: 1.3932x; 1.3726x over previous
import jax
import jax.numpy as jnp
from jax.experimental import pallas as pl
import jax.experimental.pallas.tpu as pltpu


def _lane_reduce_sum(t):
    n = t.shape[1]
    acc = t[:, 0:128]
    for k in range(1, n // 128):
        acc = acc + t[:, k * 128:(k + 1) * 128]
    return jnp.sum(acc, axis=1, keepdims=True)


def _exp_rowsum_kernel(x_ref, s_ref, rinv_ref):
    s = jnp.exp(x_ref[...] * 50.0)
    s_ref[...] = s
    rinv_ref[...] = 1.0 / _lane_reduce_sum(s)


def _col_update_kernel(s_ref, r_ref, c_ref, acc_ref):
    i = pl.program_id(0)
    ni = pl.num_programs(0)
    w = s_ref[...] * r_ref[...]

    @pl.when(i == 0)
    def _():
        acc_ref[...] = w

    @pl.when(i != 0)
    def _():
        acc_ref[...] += w

    @pl.when(i == ni - 1)
    def _():
        c_ref[...] = 1.0 / jnp.sum(acc_ref[...], axis=0, keepdims=True)


def _row_update_kernel(s_ref, c_ref, r_ref):
    r_ref[...] = 1.0 / _lane_reduce_sum(s_ref[...] * c_ref[...])


def _final_kernel(s_ref, r_ref, out_ref, idx_ref, bv_ref, bi_ref):
    j = pl.program_id(0)
    nj = pl.num_programs(0)
    m, cb = s_ref.shape
    sr = s_ref[...] * r_ref[...]
    c = 1.0 / jnp.sum(sr, axis=0, keepdims=True)
    o = sr * c
    out_ref[...] = o
    bm = jnp.max(o, axis=1, keepdims=True)
    bi = jnp.argmax(o, axis=1).reshape(m, 1).astype(jnp.int32) + j * cb

    @pl.when(j == 0)
    def _():
        bv_ref[...] = bm
        bi_ref[...] = bi

    @pl.when(j != 0)
    def _():
        upd = bm > bv_ref[...]
        bv_ref[...] = jnp.where(upd, bm, bv_ref[...])
        bi_ref[...] = jnp.where(upd, bi, bi_ref[...])

    @pl.when(j == nj - 1)
    def _():
        idx_ref[...] = bi_ref[...]


def kernel(sims, batch_size=256):
    del batch_size
    num_row, num_col = sims.shape
    work = sims.T if num_row >= num_col else sims
    m, n = work.shape

    pb = min(256, m)
    cb = min(512, n)

    s_mat, r = pl.pallas_call(
        _exp_rowsum_kernel,
        grid=(m // pb,),
        in_specs=[pl.BlockSpec((pb, n), lambda i: (i, 0))],
        out_specs=[
            pl.BlockSpec((pb, n), lambda i: (i, 0)),
            pl.BlockSpec((pb, 1), lambda i: (i, 0)),
        ],
        out_shape=[
            jax.ShapeDtypeStruct((m, n), jnp.float32),
            jax.ShapeDtypeStruct((m, 1), jnp.float32),
        ],
    )(work)

    col_update = pl.pallas_call(
        _col_update_kernel,
        grid=(m // pb,),
        in_specs=[
            pl.BlockSpec((pb, n), lambda i: (i, 0)),
            pl.BlockSpec((pb, 1), lambda i: (i, 0)),
        ],
        out_specs=pl.BlockSpec((1, n), lambda i: (0, 0)),
        out_shape=jax.ShapeDtypeStruct((1, n), jnp.float32),
        scratch_shapes=[pltpu.VMEM((pb, n), jnp.float32)],
    )

    row_update = pl.pallas_call(
        _row_update_kernel,
        grid=(m // pb,),
        in_specs=[
            pl.BlockSpec((pb, n), lambda i: (i, 0)),
            pl.BlockSpec((1, n), lambda i: (0, 0)),
        ],
        out_specs=pl.BlockSpec((pb, 1), lambda i: (i, 0)),
        out_shape=jax.ShapeDtypeStruct((m, 1), jnp.float32),
    )

    for _ in range(4):
        c = col_update(s_mat, r)
        r = row_update(s_mat, c)

    out, idx = pl.pallas_call(
        _final_kernel,
        grid=(n // cb,),
        in_specs=[
            pl.BlockSpec((m, cb), lambda j: (0, j)),
            pl.BlockSpec((m, 1), lambda j: (0, 0)),
        ],
        out_specs=[
            pl.BlockSpec((m, cb), lambda j: (0, j)),
            pl.BlockSpec((m, 1), lambda j: (0, 0)),
        ],
        out_shape=[
            jax.ShapeDtypeStruct((m, n), jnp.float32),
            jax.ShapeDtypeStruct((m, 1), jnp.int32),
        ],
        scratch_shapes=[
            pltpu.VMEM((m, 1), jnp.float32),
            pltpu.VMEM((m, 1), jnp.int32),
        ],
    )(s_mat, r)

    row_ids = jnp.arange(m, dtype=jnp.int32)
    col_ids = idx.reshape(m)
    if num_row >= num_col:
        indices = jnp.stack((col_ids, row_ids), axis=0)
    else:
        indices = jnp.stack((row_ids, col_ids), axis=0)
    values = jnp.ones((m,), dtype=jnp.float32)
    return (out, indices, values)
